# baseline (speedup 1.0000x reference)
import functools

import jax
import jax.numpy as jnp
from jax import lax
from jax.experimental import pallas as pl
from jax.experimental.pallas import tpu as pltpu
from jax.experimental.pallas import tpu_sc as plsc

N_NODES = 10000
N_EDGES = 320000
HIDDEN = 128
HALF = HIDDEN // 2
EPS = 1e-7
NC = 2
NS = 16
L = 16
CHUNK = 80
EPADDED = 327680
N_CHUNKS = 256
E_PER_W = EPADDED // NS
NPAD = 10240
ROWS_PER_W = NPAD // NS
EP = EPADDED // 2


def _emb_body(x_ref, w_ref, b_ref, g_ref, bb_ref, o_ref):
    h = jnp.dot(x_ref[...], w_ref[...], preferred_element_type=jnp.float32)
    h = h + b_ref[...]
    mu = jnp.mean(h, axis=1, keepdims=True)
    var = jnp.mean((h - mu) ** 2, axis=1, keepdims=True)
    h = (h - mu) / jnp.sqrt(var + 1e-5) * g_ref[...] + bb_ref[...]
    o_ref[...] = jnp.maximum(h, 0.0)


def _node_embed(x2, W_emb, b_emb, ln_g, ln_b):
    return pl.pallas_call(
        _emb_body,
        out_shape=jax.ShapeDtypeStruct((N_NODES, HIDDEN), jnp.float32),
    )(x2, W_emb, b_emb, ln_g, ln_b)


_BE = 2048


def _edge_body(a_ref, w_ref, b_ref, o_ref):
    a = a_ref[...]
    for c in range(NC):
        e = jnp.dot(a, w_ref[c], preferred_element_type=jnp.float32)
        o_ref[c] = e + b_ref[c]


def _edge_embed(attr2, W2, b2):
    return pl.pallas_call(
        _edge_body,
        grid=(EP // _BE,),
        in_specs=[
            pl.BlockSpec((_BE, 32), lambda i: (i, 0)),
            pl.BlockSpec((NC, 32, HIDDEN), lambda i: (0, 0, 0)),
            pl.BlockSpec((NC, HIDDEN), lambda i: (0, 0)),
        ],
        out_specs=pl.BlockSpec((NC, _BE, HIDDEN), lambda i: (0, i, 0)),
        out_shape=jax.ShapeDtypeStruct((NC, EP, HIDDEN), jnp.float32),
    )(attr2, W2, b2)


def _sc_aggregate_body(h_hbm, ep_hbm, src_hbm, dst_hbm, z_hbm, out_hbm,
                       acc, src_a, dst_a, src_b, dst_b,
                       hs_a, hs_b, e_a, e_b, out_v,
                       gsem_a, gsem_b, esem_a, esem_b, isem_a, isem_b):
    c = lax.axis_index("c")
    s = lax.axis_index("s")
    pltpu.sync_copy(z_hbm, acc.at[pl.ds(s * ROWS_PER_W, ROWS_PER_W)])
    plsc.subcore_barrier()

    e_base0 = s * E_PER_W
    ep0 = c * EP + s * (E_PER_W // 2)
    coff = c * HALF

    def load_idx(i, sv, dv, isem):
        base = e_base0 + i * CHUNK
        pltpu.async_copy(src_hbm.at[pl.ds(base, CHUNK)], sv, isem)
        pltpu.async_copy(dst_hbm.at[pl.ds(base, CHUNK)], dv, isem)

    def wait_idx(i, sv, dv, isem):
        base = e_base0 + i * CHUNK
        pltpu.make_async_copy(src_hbm.at[pl.ds(base, CHUNK)], sv, isem).wait()
        pltpu.make_async_copy(dst_hbm.at[pl.ds(base, CHUNK)], dv, isem).wait()

    HC = CHUNK // 2

    def load(i, sv, hs, eb, gsem, esem):
        pltpu.async_copy(h_hbm.at[sv.at[pl.ds(0, HC)]], hs.at[pl.ds(0, HC)], gsem)
        pltpu.async_copy(h_hbm.at[sv.at[pl.ds(HC, HC)]], hs.at[pl.ds(HC, HC)], gsem)
        pltpu.async_copy(ep_hbm.at[pl.ds(ep0 + i * (CHUNK // 2), CHUNK // 2)],
                         eb, esem)

    def wait_load(i, sv, hs, eb, gsem, esem):
        pltpu.make_async_copy(h_hbm.at[sv.at[pl.ds(0, HC)]], hs.at[pl.ds(0, HC)], gsem).wait()
        pltpu.make_async_copy(h_hbm.at[sv.at[pl.ds(HC, HC)]], hs.at[pl.ds(HC, HC)], gsem).wait()
        pltpu.make_async_copy(ep_hbm.at[pl.ds(ep0 + i * (CHUNK // 2), CHUNK // 2)],
                              eb, esem).wait()

    def compute_scatter(dv, hs, eb):
        nq = HALF // L

        def pair_body(jp, carry2):
            keys = [(2 * jp + half, half, q) for half in range(2)
                    for q in range(nq)]
            hv = [hs[j, pl.ds(coff + q * L, L)] for (j, _, q) in keys]
            ev = [eb[jp, pl.ds(half * HALF + q * L, L)]
                  for (_, half, q) in keys]
            t = [a + b for a, b in zip(hv, ev)]
            t = [jnp.maximum(a, 0.0) for a in t]
            ms = [a + EPS for a in t]
            exs = [jnp.exp(a) for a in ms]
            prs = [a * b for a, b in zip(ms, exs)]
            for k, (j, _, q) in enumerate(keys):
                out_v[j, pl.ds(q * L, L)] = prs[k]
            for k, (j, _, q) in enumerate(keys):
                out_v[j, pl.ds(HALF + q * L, L)] = exs[k]
            return carry2

        lax.fori_loop(0, CHUNK // 2, pair_body, 0)
        pltpu.sync_copy(out_v, acc.at[dv], add=True)

    load_idx(0, src_a, dst_a, isem_a)
    wait_idx(0, src_a, dst_a, isem_a)
    load(0, src_a, hs_a, e_a, gsem_a, esem_a)
    load_idx(1, src_b, dst_b, isem_b)

    def body(t, carry):
        i0 = 2 * t
        i1 = i0 + 1
        wait_idx(i1, src_b, dst_b, isem_b)
        load(i1, src_b, hs_b, e_b, gsem_b, esem_b)
        wait_load(i0, src_a, hs_a, e_a, gsem_a, esem_a)
        compute_scatter(dst_a, hs_a, e_a)

        @pl.when(t < N_CHUNKS // 2 - 1)
        def _():
            load_idx(i0 + 2, src_a, dst_a, isem_a)

        wait_load(i1, src_b, hs_b, e_b, gsem_b, esem_b)
        compute_scatter(dst_b, hs_b, e_b)

        @pl.when(t < N_CHUNKS // 2 - 1)
        def _():
            load_idx(i1 + 2, src_b, dst_b, isem_b)
            wait_idx(i0 + 2, src_a, dst_a, isem_a)
            load(i0 + 2, src_a, hs_a, e_a, gsem_a, esem_a)

        return carry

    lax.fori_loop(0, N_CHUNKS // 2, body, 0)
    plsc.subcore_barrier()
    pltpu.sync_copy(
        acc.at[pl.ds(s * ROWS_PER_W, ROWS_PER_W)],
        out_hbm.at[pl.ds(c * NPAD + s * ROWS_PER_W, ROWS_PER_W)],
    )


@functools.lru_cache(maxsize=1)
def _make_sc_aggregate():
    return functools.partial(
        pl.kernel,
        out_type=jax.ShapeDtypeStruct((NC * NPAD, HIDDEN), jnp.float32),
        mesh=plsc.VectorSubcoreMesh(
            core_axis_name="c", subcore_axis_name="s",
            num_cores=NC, num_subcores=NS,
        ),
        scratch_types=[
            pltpu.VMEM_SHARED((NPAD, HIDDEN), jnp.float32),
            pltpu.VMEM((CHUNK,), jnp.int32),
            pltpu.VMEM((CHUNK,), jnp.int32),
            pltpu.VMEM((CHUNK,), jnp.int32),
            pltpu.VMEM((CHUNK,), jnp.int32),
            pltpu.VMEM((CHUNK, HIDDEN), jnp.float32),
            pltpu.VMEM((CHUNK, HIDDEN), jnp.float32),
            pltpu.VMEM((CHUNK // 2, HIDDEN), jnp.float32),
            pltpu.VMEM((CHUNK // 2, HIDDEN), jnp.float32),
            pltpu.VMEM((CHUNK, HIDDEN), jnp.float32),
            pltpu.SemaphoreType.DMA,
            pltpu.SemaphoreType.DMA,
            pltpu.SemaphoreType.DMA,
            pltpu.SemaphoreType.DMA,
            pltpu.SemaphoreType.DMA,
            pltpu.SemaphoreType.DMA,
        ],
    )(_sc_aggregate_body)


def _sc_aggregate(h, ep, src, dst, zrows):
    return _make_sc_aggregate()(h, ep, src, dst, zrows)


def _mlp_body(h_ref, acc_ref,
              w0, b0, w1, b1, w2, b2, w3, b3, w4, b4,
              g0, bb0, g1, bb1, g2, bb2, g3, bb3,
              wd, bd, o_ref):
    h = h_ref[...]
    p = jnp.concatenate([acc_ref[0][:, :HALF], acc_ref[1][:, :HALF]], axis=1)
    dn = jnp.concatenate([acc_ref[0][:, HALF:], acc_ref[1][:, HALF:]], axis=1)
    z = p / (dn + 1e-16) + h
    ws = (w0, w1, w2, w3, w4)
    bs = (b0, b1, b2, b3, b4)
    gs = (g0, g1, g2, g3)
    bbs = (bb0, bb1, bb2, bb3)
    for i in range(5):
        z = jnp.dot(z, ws[i][...], preferred_element_type=jnp.float32) + bs[i][...]
        if i < 4:
            mu = jnp.mean(z, axis=0, keepdims=True)
            var = jnp.mean((z - mu) ** 2, axis=0, keepdims=True)
            z = (z - mu) / jnp.sqrt(var + 1e-5) * gs[i][...] + bbs[i][...]
            z = jnp.maximum(z, 0.0)
    z = jnp.maximum(z, 0.0)
    z = jnp.dot(z, wd[...], preferred_element_type=jnp.float32) + bd[...]
    o_ref[...] = z


def _combine_mlp(h, acc, mlp_Ws, mlp_bs, bn_gs, bn_bs, W_dec, b_dec):
    args = [h, acc]
    for w, b in zip(mlp_Ws, mlp_bs):
        args += [w, b]
    for g, b in zip(bn_gs, bn_bs):
        args += [g, b]
    args += [W_dec, b_dec]
    return pl.pallas_call(
        _mlp_body,
        out_shape=jax.ShapeDtypeStruct((N_NODES, HIDDEN), jnp.float32),
    )(*args)


def kernel(x, edge_index, edge_attr, W_emb, b_emb, ln_g, ln_b, W_edge, b_edge,
           mlp_W0, mlp_b0, mlp_W1, mlp_b1, mlp_W2, mlp_b2, mlp_W3, mlp_b3,
           mlp_W4, mlp_b4, bn_g0, bn_b0, bn_g1, bn_b1, bn_g2, bn_b2, bn_g3,
           bn_b3, W_dec, b_dec):
    x2 = x.reshape(N_NODES, -1)
    h = _node_embed(x2, W_emb, b_emb, ln_g, ln_b)

    zero16 = jnp.zeros((16, HALF), jnp.float32)
    W2 = jnp.stack([
        jnp.concatenate([
            jnp.concatenate([W_edge[:, c * HALF:(c + 1) * HALF], zero16], axis=1),
            jnp.concatenate([zero16, W_edge[:, c * HALF:(c + 1) * HALF]], axis=1),
        ], axis=0)
        for c in range(NC)
    ])
    b2 = jnp.stack([
        jnp.concatenate([b_edge[c * HALF:(c + 1) * HALF]] * 2)
        for c in range(NC)
    ])
    npad_e = EPADDED - N_EDGES
    attr2 = jnp.pad(edge_attr.reshape(N_EDGES // 2, 32), ((0, npad_e // 2), (0, 0)))
    ep = _edge_embed(attr2, W2, b2)

    src = jnp.pad(edge_index[0], (0, npad_e))
    dst = jnp.pad(edge_index[1], (0, npad_e), constant_values=NPAD - 1)
    zrows = jnp.zeros((ROWS_PER_W, HIDDEN), jnp.float32)
    acc = _sc_aggregate(h, ep.reshape(NC * EP, HIDDEN), src, dst, zrows)
    acc = acc.reshape(NC, NPAD, HIDDEN)[:, :N_NODES, :]
    z = _combine_mlp(h, acc,
                     (mlp_W0, mlp_W1, mlp_W2, mlp_W3, mlp_W4),
                     (mlp_b0, mlp_b1, mlp_b2, mlp_b3, mlp_b4),
                     (bn_g0, bn_g1, bn_g2, bn_g3),
                     (bn_b0, bn_b1, bn_b2, bn_b3),
                     W_dec, b_dec)
    return z.reshape(N_NODES, HIDDEN // 2, 2)


# speedup vs baseline: 1.3199x; 1.3199x over previous
import functools

import jax
import jax.numpy as jnp
from jax import lax
from jax.experimental import pallas as pl
from jax.experimental.pallas import tpu as pltpu
from jax.experimental.pallas import tpu_sc as plsc

N_NODES = 10000
N_EDGES = 320000
HIDDEN = 128
HALF = HIDDEN // 2
EPS = 1e-7
NC = 2
NS = 16
L = 16
CHUNK = 80
EPADDED = 327680
N_CHUNKS = 256
E_PER_W = EPADDED // NS
NPAD = 10240
ROWS_PER_W = NPAD // NS
EP = EPADDED // 2


def _emb_body(x_ref, w_ref, b_ref, g_ref, bb_ref, o_ref, osp_ref):
    h = jnp.dot(x_ref[...], w_ref[...], preferred_element_type=jnp.float32)
    h = h + b_ref[...]
    mu = jnp.mean(h, axis=1, keepdims=True)
    var = jnp.mean((h - mu) ** 2, axis=1, keepdims=True)
    h = (h - mu) / jnp.sqrt(var + 1e-5) * g_ref[...] + bb_ref[...]
    h = jnp.maximum(h, 0.0)
    o_ref[...] = h
    osp_ref[0] = h[:, :HALF]
    osp_ref[1] = h[:, HALF:]


def _node_embed(x2, W_emb, b_emb, ln_g, ln_b):
    return pl.pallas_call(
        _emb_body,
        out_shape=[jax.ShapeDtypeStruct((N_NODES, HIDDEN), jnp.float32),
                   jax.ShapeDtypeStruct((2, N_NODES, HALF), jnp.float32)],
    )(x2, W_emb, b_emb, ln_g, ln_b)


_BE = 2048


def _edge_body(a_ref, w_ref, b_ref, o_ref):
    a = a_ref[...]
    for c in range(NC):
        e = jnp.dot(a, w_ref[c], preferred_element_type=jnp.float32)
        o_ref[c] = e + b_ref[c]


def _edge_embed(attr2, W2, b2):
    return pl.pallas_call(
        _edge_body,
        grid=(EP // _BE,),
        in_specs=[
            pl.BlockSpec((_BE, 32), lambda i: (i, 0)),
            pl.BlockSpec((NC, 32, HIDDEN), lambda i: (0, 0, 0)),
            pl.BlockSpec((NC, HIDDEN), lambda i: (0, 0)),
        ],
        out_specs=pl.BlockSpec((NC, _BE, HIDDEN), lambda i: (0, i, 0)),
        out_shape=jax.ShapeDtypeStruct((NC, EP, HIDDEN), jnp.float32),
    )(attr2, W2, b2)


def _sc_aggregate_body(h_hbm, ep_hbm, src_hbm, dst_hbm, z_hbm, out_hbm,
                       acc, src_a, dst_a, src_b, dst_b,
                       hs_a, hs_b, e_a, e_b, out_v,
                       gsem_a, gsem_b, esem_a, esem_b, isem_a, isem_b):
    c = lax.axis_index("c")
    s = lax.axis_index("s")
    pltpu.sync_copy(z_hbm, acc.at[pl.ds(s * ROWS_PER_W, ROWS_PER_W)])
    plsc.subcore_barrier()

    e_base0 = s * E_PER_W
    ep0 = c * EP + s * (E_PER_W // 2)
    coff_h = c * N_NODES

    def load_idx(i, sv, dv, isem):
        base = e_base0 + i * CHUNK
        pltpu.async_copy(src_hbm.at[pl.ds(base, CHUNK)], sv, isem)
        pltpu.async_copy(dst_hbm.at[pl.ds(base, CHUNK)], dv, isem)

    def wait_idx(i, sv, dv, isem):
        base = e_base0 + i * CHUNK
        pltpu.make_async_copy(src_hbm.at[pl.ds(base, CHUNK)], sv, isem).wait()
        pltpu.make_async_copy(dst_hbm.at[pl.ds(base, CHUNK)], dv, isem).wait()
        for k in range(CHUNK // L):
            sv[pl.ds(k * L, L)] = sv[pl.ds(k * L, L)] + coff_h

    def load(i, sv, hs, eb, gsem, esem):
        pltpu.async_copy(h_hbm.at[sv], hs, gsem)
        pltpu.async_copy(ep_hbm.at[pl.ds(ep0 + i * (CHUNK // 2), CHUNK // 2)],
                         eb, esem)

    def wait_load(i, sv, hs, eb, gsem, esem):
        pltpu.make_async_copy(h_hbm.at[sv], hs, gsem).wait()
        pltpu.make_async_copy(ep_hbm.at[pl.ds(ep0 + i * (CHUNK // 2), CHUNK // 2)],
                              eb, esem).wait()

    def compute_scatter(dv, hs, eb):
        nq = HALF // L

        def pair_body(jp, carry2):
            keys = [(2 * jp + half, half, q) for half in range(2)
                    for q in range(nq)]
            hv = [hs[j, pl.ds(q * L, L)] for (j, _, q) in keys]
            ev = [eb[jp, pl.ds(half * HALF + q * L, L)]
                  for (_, half, q) in keys]
            t = [a + b for a, b in zip(hv, ev)]
            t = [jnp.maximum(a, 0.0) for a in t]
            ms = [a + EPS for a in t]
            exs = [jnp.exp(a) for a in ms]
            prs = [a * b for a, b in zip(ms, exs)]
            for k, (j, _, q) in enumerate(keys):
                out_v[j, pl.ds(q * L, L)] = prs[k]
            for k, (j, _, q) in enumerate(keys):
                out_v[j, pl.ds(HALF + q * L, L)] = exs[k]
            return carry2

        lax.fori_loop(0, CHUNK // 2, pair_body, 0)
        pltpu.sync_copy(out_v, acc.at[dv], add=True)

    load_idx(0, src_a, dst_a, isem_a)
    wait_idx(0, src_a, dst_a, isem_a)
    load(0, src_a, hs_a, e_a, gsem_a, esem_a)
    load_idx(1, src_b, dst_b, isem_b)

    def body(t, carry):
        i0 = 2 * t
        i1 = i0 + 1
        wait_idx(i1, src_b, dst_b, isem_b)
        load(i1, src_b, hs_b, e_b, gsem_b, esem_b)
        wait_load(i0, src_a, hs_a, e_a, gsem_a, esem_a)
        compute_scatter(dst_a, hs_a, e_a)

        @pl.when(t < N_CHUNKS // 2 - 1)
        def _():
            load_idx(i0 + 2, src_a, dst_a, isem_a)

        wait_load(i1, src_b, hs_b, e_b, gsem_b, esem_b)
        compute_scatter(dst_b, hs_b, e_b)

        @pl.when(t < N_CHUNKS // 2 - 1)
        def _():
            load_idx(i1 + 2, src_b, dst_b, isem_b)
            wait_idx(i0 + 2, src_a, dst_a, isem_a)
            load(i0 + 2, src_a, hs_a, e_a, gsem_a, esem_a)

        return carry

    lax.fori_loop(0, N_CHUNKS // 2, body, 0)
    plsc.subcore_barrier()
    pltpu.sync_copy(
        acc.at[pl.ds(s * ROWS_PER_W, ROWS_PER_W)],
        out_hbm.at[pl.ds(c * NPAD + s * ROWS_PER_W, ROWS_PER_W)],
    )


@functools.lru_cache(maxsize=1)
def _make_sc_aggregate():
    return functools.partial(
        pl.kernel,
        out_type=jax.ShapeDtypeStruct((NC * NPAD, HIDDEN), jnp.float32),
        mesh=plsc.VectorSubcoreMesh(
            core_axis_name="c", subcore_axis_name="s",
            num_cores=NC, num_subcores=NS,
        ),
        compiler_params=pltpu.CompilerParams(use_tc_tiling_on_sc=False),
        scratch_types=[
            pltpu.VMEM_SHARED((NPAD, HIDDEN), jnp.float32),
            pltpu.VMEM((CHUNK,), jnp.int32),
            pltpu.VMEM((CHUNK,), jnp.int32),
            pltpu.VMEM((CHUNK,), jnp.int32),
            pltpu.VMEM((CHUNK,), jnp.int32),
            pltpu.VMEM((CHUNK, HALF), jnp.float32),
            pltpu.VMEM((CHUNK, HALF), jnp.float32),
            pltpu.VMEM((CHUNK // 2, HIDDEN), jnp.float32),
            pltpu.VMEM((CHUNK // 2, HIDDEN), jnp.float32),
            pltpu.VMEM((CHUNK, HIDDEN), jnp.float32),
            pltpu.SemaphoreType.DMA,
            pltpu.SemaphoreType.DMA,
            pltpu.SemaphoreType.DMA,
            pltpu.SemaphoreType.DMA,
            pltpu.SemaphoreType.DMA,
            pltpu.SemaphoreType.DMA,
        ],
    )(_sc_aggregate_body)


def _sc_aggregate(h, ep, src, dst, zrows):
    return _make_sc_aggregate()(h, ep, src, dst, zrows)


def _mlp_body(h_ref, acc_ref,
              w0, b0, w1, b1, w2, b2, w3, b3, w4, b4,
              g0, bb0, g1, bb1, g2, bb2, g3, bb3,
              wd, bd, o_ref):
    h = h_ref[...]
    p = jnp.concatenate([acc_ref[0][:, :HALF], acc_ref[1][:, :HALF]], axis=1)
    dn = jnp.concatenate([acc_ref[0][:, HALF:], acc_ref[1][:, HALF:]], axis=1)
    z = p / (dn + 1e-16) + h
    ws = (w0, w1, w2, w3, w4)
    bs = (b0, b1, b2, b3, b4)
    gs = (g0, g1, g2, g3)
    bbs = (bb0, bb1, bb2, bb3)
    for i in range(5):
        z = jnp.dot(z, ws[i][...], preferred_element_type=jnp.float32) + bs[i][...]
        if i < 4:
            mu = jnp.mean(z, axis=0, keepdims=True)
            var = jnp.mean((z - mu) ** 2, axis=0, keepdims=True)
            z = (z - mu) / jnp.sqrt(var + 1e-5) * gs[i][...] + bbs[i][...]
            z = jnp.maximum(z, 0.0)
    z = jnp.maximum(z, 0.0)
    z = jnp.dot(z, wd[...], preferred_element_type=jnp.float32) + bd[...]
    o_ref[...] = z


def _combine_mlp(h, acc, mlp_Ws, mlp_bs, bn_gs, bn_bs, W_dec, b_dec):
    args = [h, acc]
    for w, b in zip(mlp_Ws, mlp_bs):
        args += [w, b]
    for g, b in zip(bn_gs, bn_bs):
        args += [g, b]
    args += [W_dec, b_dec]
    return pl.pallas_call(
        _mlp_body,
        out_shape=jax.ShapeDtypeStruct((N_NODES, HIDDEN), jnp.float32),
    )(*args)


def kernel(x, edge_index, edge_attr, W_emb, b_emb, ln_g, ln_b, W_edge, b_edge,
           mlp_W0, mlp_b0, mlp_W1, mlp_b1, mlp_W2, mlp_b2, mlp_W3, mlp_b3,
           mlp_W4, mlp_b4, bn_g0, bn_b0, bn_g1, bn_b1, bn_g2, bn_b2, bn_g3,
           bn_b3, W_dec, b_dec):
    x2 = x.reshape(N_NODES, -1)
    h, hsp = _node_embed(x2, W_emb, b_emb, ln_g, ln_b)

    zero16 = jnp.zeros((16, HALF), jnp.float32)
    W2 = jnp.stack([
        jnp.concatenate([
            jnp.concatenate([W_edge[:, c * HALF:(c + 1) * HALF], zero16], axis=1),
            jnp.concatenate([zero16, W_edge[:, c * HALF:(c + 1) * HALF]], axis=1),
        ], axis=0)
        for c in range(NC)
    ])
    b2 = jnp.stack([
        jnp.concatenate([b_edge[c * HALF:(c + 1) * HALF]] * 2)
        for c in range(NC)
    ])
    npad_e = EPADDED - N_EDGES
    attr2 = jnp.pad(edge_attr.reshape(N_EDGES // 2, 32), ((0, npad_e // 2), (0, 0)))
    ep = _edge_embed(attr2, W2, b2)

    src = jnp.pad(edge_index[0], (0, npad_e))
    dst = jnp.pad(edge_index[1], (0, npad_e), constant_values=NPAD - 1)
    zrows = jnp.zeros((ROWS_PER_W, HIDDEN), jnp.float32)
    acc = _sc_aggregate(hsp.reshape(NC * N_NODES, HALF),
                        ep.reshape(NC * EP, HIDDEN), src, dst, zrows)
    acc = acc.reshape(NC, NPAD, HIDDEN)[:, :N_NODES, :]
    z = _combine_mlp(h, acc,
                     (mlp_W0, mlp_W1, mlp_W2, mlp_W3, mlp_W4),
                     (mlp_b0, mlp_b1, mlp_b2, mlp_b3, mlp_b4),
                     (bn_g0, bn_g1, bn_g2, bn_g3),
                     (bn_b0, bn_b1, bn_b2, bn_b3),
                     W_dec, b_dec)
    return z.reshape(N_NODES, HIDDEN // 2, 2)


# speedup vs baseline: 1.3627x; 1.0324x over previous
import functools

import jax
import jax.numpy as jnp
from jax import lax
from jax.experimental import pallas as pl
from jax.experimental.pallas import tpu as pltpu
from jax.experimental.pallas import tpu_sc as plsc

N_NODES = 10000
N_EDGES = 320000
HIDDEN = 128
HALF = HIDDEN // 2
EPS = 1e-7
NC = 2
NS = 16
L = 16
CHUNK = 128
EPADDED = 327680
N_CHUNKS = 160
E_PER_W = EPADDED // NS
NPAD = 10112
ROWS_PER_W = NPAD // NS
EP = EPADDED // 2


def _emb_body(x_ref, w_ref, b_ref, g_ref, bb_ref, o_ref, osp_ref):
    h = jnp.dot(x_ref[...], w_ref[...], preferred_element_type=jnp.float32)
    h = h + b_ref[...]
    mu = jnp.mean(h, axis=1, keepdims=True)
    var = jnp.mean((h - mu) ** 2, axis=1, keepdims=True)
    h = (h - mu) / jnp.sqrt(var + 1e-5) * g_ref[...] + bb_ref[...]
    h = jnp.maximum(h, 0.0)
    o_ref[...] = h
    osp_ref[0] = h[:, :HALF]
    osp_ref[1] = h[:, HALF:]


def _node_embed(x2, W_emb, b_emb, ln_g, ln_b):
    return pl.pallas_call(
        _emb_body,
        out_shape=[jax.ShapeDtypeStruct((N_NODES, HIDDEN), jnp.float32),
                   jax.ShapeDtypeStruct((2, N_NODES, HALF), jnp.float32)],
    )(x2, W_emb, b_emb, ln_g, ln_b)


_BE = 2048


def _edge_body(a_ref, w_ref, b_ref, o_ref):
    a = a_ref[...]
    for c in range(NC):
        e = jnp.dot(a, w_ref[c], preferred_element_type=jnp.float32)
        o_ref[c] = e + b_ref[c]


def _edge_embed(attr2, W2, b2):
    return pl.pallas_call(
        _edge_body,
        grid=(EP // _BE,),
        in_specs=[
            pl.BlockSpec((_BE, 32), lambda i: (i, 0)),
            pl.BlockSpec((NC, 32, HIDDEN), lambda i: (0, 0, 0)),
            pl.BlockSpec((NC, HIDDEN), lambda i: (0, 0)),
        ],
        out_specs=pl.BlockSpec((NC, _BE, HIDDEN), lambda i: (0, i, 0)),
        out_shape=jax.ShapeDtypeStruct((NC, EP, HIDDEN), jnp.float32),
    )(attr2, W2, b2)


def _sc_aggregate_body(h_hbm, ep_hbm, src_hbm, dst_hbm, z_hbm, out_hbm,
                       acc, src_a, dst_a, src_b, dst_b,
                       hs_a, hs_b, e_a, e_b, out_v,
                       gsem_a, gsem_b, esem_a, esem_b, isem_a, isem_b):
    c = lax.axis_index("c")
    s = lax.axis_index("s")
    pltpu.sync_copy(z_hbm, acc.at[pl.ds(s * ROWS_PER_W, ROWS_PER_W)])
    plsc.subcore_barrier()

    e_base0 = s * E_PER_W
    ep0 = c * EP + s * (E_PER_W // 2)
    coff_h = c * N_NODES

    def load_idx(i, sv, dv, isem):
        base = e_base0 + i * CHUNK
        pltpu.async_copy(src_hbm.at[pl.ds(base, CHUNK)], sv, isem)
        pltpu.async_copy(dst_hbm.at[pl.ds(base, CHUNK)], dv, isem)

    def wait_idx(i, sv, dv, isem):
        base = e_base0 + i * CHUNK
        pltpu.make_async_copy(src_hbm.at[pl.ds(base, CHUNK)], sv, isem).wait()
        pltpu.make_async_copy(dst_hbm.at[pl.ds(base, CHUNK)], dv, isem).wait()
        for k in range(CHUNK // L):
            sv[pl.ds(k * L, L)] = sv[pl.ds(k * L, L)] + coff_h

    def load(i, sv, hs, eb, gsem, esem):
        pltpu.async_copy(h_hbm.at[sv], hs, gsem)
        pltpu.async_copy(ep_hbm.at[pl.ds(ep0 + i * (CHUNK // 2), CHUNK // 2)],
                         eb, esem)

    def wait_load(i, sv, hs, eb, gsem, esem):
        pltpu.make_async_copy(h_hbm.at[sv], hs, gsem).wait()
        pltpu.make_async_copy(ep_hbm.at[pl.ds(ep0 + i * (CHUNK // 2), CHUNK // 2)],
                              eb, esem).wait()

    def compute_scatter(dv, hs, eb):
        nq = HALF // L

        def pair_body(jp, carry2):
            keys = [(2 * jp + half, half, q) for half in range(2)
                    for q in range(nq)]
            hv = [hs[j, pl.ds(q * L, L)] for (j, _, q) in keys]
            ev = [eb[jp, pl.ds(half * HALF + q * L, L)]
                  for (_, half, q) in keys]
            t = [a + b for a, b in zip(hv, ev)]
            t = [jnp.maximum(a, 0.0) for a in t]
            ms = [a + EPS for a in t]
            exs = [jnp.exp(a) for a in ms]
            prs = [a * b for a, b in zip(ms, exs)]
            for k, (j, _, q) in enumerate(keys):
                out_v[j, pl.ds(q * L, L)] = prs[k]
            for k, (j, _, q) in enumerate(keys):
                out_v[j, pl.ds(HALF + q * L, L)] = exs[k]
            return carry2

        lax.fori_loop(0, CHUNK // 2, pair_body, 0)
        pltpu.sync_copy(out_v, acc.at[dv], add=True)

    load_idx(0, src_a, dst_a, isem_a)
    wait_idx(0, src_a, dst_a, isem_a)
    load(0, src_a, hs_a, e_a, gsem_a, esem_a)
    load_idx(1, src_b, dst_b, isem_b)

    def body(t, carry):
        i0 = 2 * t
        i1 = i0 + 1
        wait_idx(i1, src_b, dst_b, isem_b)
        load(i1, src_b, hs_b, e_b, gsem_b, esem_b)
        wait_load(i0, src_a, hs_a, e_a, gsem_a, esem_a)
        compute_scatter(dst_a, hs_a, e_a)

        @pl.when(t < N_CHUNKS // 2 - 1)
        def _():
            load_idx(i0 + 2, src_a, dst_a, isem_a)

        wait_load(i1, src_b, hs_b, e_b, gsem_b, esem_b)
        compute_scatter(dst_b, hs_b, e_b)

        @pl.when(t < N_CHUNKS // 2 - 1)
        def _():
            load_idx(i1 + 2, src_b, dst_b, isem_b)
            wait_idx(i0 + 2, src_a, dst_a, isem_a)
            load(i0 + 2, src_a, hs_a, e_a, gsem_a, esem_a)

        return carry

    lax.fori_loop(0, N_CHUNKS // 2, body, 0)
    plsc.subcore_barrier()
    pltpu.sync_copy(
        acc.at[pl.ds(s * ROWS_PER_W, ROWS_PER_W)],
        out_hbm.at[pl.ds(c * NPAD + s * ROWS_PER_W, ROWS_PER_W)],
    )


@functools.lru_cache(maxsize=1)
def _make_sc_aggregate():
    return functools.partial(
        pl.kernel,
        out_type=jax.ShapeDtypeStruct((NC * NPAD, HIDDEN), jnp.float32),
        mesh=plsc.VectorSubcoreMesh(
            core_axis_name="c", subcore_axis_name="s",
            num_cores=NC, num_subcores=NS,
        ),
        compiler_params=pltpu.CompilerParams(use_tc_tiling_on_sc=False),
        scratch_types=[
            pltpu.VMEM_SHARED((NPAD, HIDDEN), jnp.float32),
            pltpu.VMEM((CHUNK,), jnp.int32),
            pltpu.VMEM((CHUNK,), jnp.int32),
            pltpu.VMEM((CHUNK,), jnp.int32),
            pltpu.VMEM((CHUNK,), jnp.int32),
            pltpu.VMEM((CHUNK, HALF), jnp.float32),
            pltpu.VMEM((CHUNK, HALF), jnp.float32),
            pltpu.VMEM((CHUNK // 2, HIDDEN), jnp.float32),
            pltpu.VMEM((CHUNK // 2, HIDDEN), jnp.float32),
            pltpu.VMEM((CHUNK, HIDDEN), jnp.float32),
            pltpu.SemaphoreType.DMA,
            pltpu.SemaphoreType.DMA,
            pltpu.SemaphoreType.DMA,
            pltpu.SemaphoreType.DMA,
            pltpu.SemaphoreType.DMA,
            pltpu.SemaphoreType.DMA,
        ],
    )(_sc_aggregate_body)


def _sc_aggregate(h, ep, src, dst, zrows):
    return _make_sc_aggregate()(h, ep, src, dst, zrows)


def _mlp_body(h_ref, acc_ref,
              w0, b0, w1, b1, w2, b2, w3, b3, w4, b4,
              g0, bb0, g1, bb1, g2, bb2, g3, bb3,
              wd, bd, o_ref):
    h = h_ref[...]
    p = jnp.concatenate([acc_ref[0][:, :HALF], acc_ref[1][:, :HALF]], axis=1)
    dn = jnp.concatenate([acc_ref[0][:, HALF:], acc_ref[1][:, HALF:]], axis=1)
    z = p / (dn + 1e-16) + h
    ws = (w0, w1, w2, w3, w4)
    bs = (b0, b1, b2, b3, b4)
    gs = (g0, g1, g2, g3)
    bbs = (bb0, bb1, bb2, bb3)
    for i in range(5):
        z = jnp.dot(z, ws[i][...], preferred_element_type=jnp.float32) + bs[i][...]
        if i < 4:
            mu = jnp.mean(z, axis=0, keepdims=True)
            var = jnp.mean((z - mu) ** 2, axis=0, keepdims=True)
            z = (z - mu) / jnp.sqrt(var + 1e-5) * gs[i][...] + bbs[i][...]
            z = jnp.maximum(z, 0.0)
    z = jnp.maximum(z, 0.0)
    z = jnp.dot(z, wd[...], preferred_element_type=jnp.float32) + bd[...]
    o_ref[...] = z


def _combine_mlp(h, acc, mlp_Ws, mlp_bs, bn_gs, bn_bs, W_dec, b_dec):
    args = [h, acc]
    for w, b in zip(mlp_Ws, mlp_bs):
        args += [w, b]
    for g, b in zip(bn_gs, bn_bs):
        args += [g, b]
    args += [W_dec, b_dec]
    return pl.pallas_call(
        _mlp_body,
        out_shape=jax.ShapeDtypeStruct((N_NODES, HIDDEN), jnp.float32),
    )(*args)


def kernel(x, edge_index, edge_attr, W_emb, b_emb, ln_g, ln_b, W_edge, b_edge,
           mlp_W0, mlp_b0, mlp_W1, mlp_b1, mlp_W2, mlp_b2, mlp_W3, mlp_b3,
           mlp_W4, mlp_b4, bn_g0, bn_b0, bn_g1, bn_b1, bn_g2, bn_b2, bn_g3,
           bn_b3, W_dec, b_dec):
    x2 = x.reshape(N_NODES, -1)
    h, hsp = _node_embed(x2, W_emb, b_emb, ln_g, ln_b)

    zero16 = jnp.zeros((16, HALF), jnp.float32)
    W2 = jnp.stack([
        jnp.concatenate([
            jnp.concatenate([W_edge[:, c * HALF:(c + 1) * HALF], zero16], axis=1),
            jnp.concatenate([zero16, W_edge[:, c * HALF:(c + 1) * HALF]], axis=1),
        ], axis=0)
        for c in range(NC)
    ])
    b2 = jnp.stack([
        jnp.concatenate([b_edge[c * HALF:(c + 1) * HALF]] * 2)
        for c in range(NC)
    ])
    npad_e = EPADDED - N_EDGES
    attr2 = jnp.pad(edge_attr.reshape(N_EDGES // 2, 32), ((0, npad_e // 2), (0, 0)))
    ep = _edge_embed(attr2, W2, b2)

    src = jnp.pad(edge_index[0], (0, npad_e))
    dst = jnp.pad(edge_index[1], (0, npad_e), constant_values=NPAD - 1)
    zrows = jnp.zeros((ROWS_PER_W, HIDDEN), jnp.float32)
    acc = _sc_aggregate(hsp.reshape(NC * N_NODES, HALF),
                        ep.reshape(NC * EP, HIDDEN), src, dst, zrows)
    acc = acc.reshape(NC, NPAD, HIDDEN)[:, :N_NODES, :]
    z = _combine_mlp(h, acc,
                     (mlp_W0, mlp_W1, mlp_W2, mlp_W3, mlp_W4),
                     (mlp_b0, mlp_b1, mlp_b2, mlp_b3, mlp_b4),
                     (bn_g0, bn_g1, bn_g2, bn_g3),
                     (bn_b0, bn_b1, bn_b2, bn_b3),
                     W_dec, b_dec)
    return z.reshape(N_NODES, HIDDEN // 2, 2)


# speedup vs baseline: 1.4430x; 1.0589x over previous
import functools

import jax
import jax.numpy as jnp
from jax import lax
from jax.experimental import pallas as pl
from jax.experimental.pallas import tpu as pltpu
from jax.experimental.pallas import tpu_sc as plsc

N_NODES = 10000
N_EDGES = 320000
HIDDEN = 128
HALF = HIDDEN // 2
EPS = 1e-7
NC = 2
NS = 16
L = 16
CHUNK = 128
EPADDED = 327680
N_CHUNKS = 160
E_PER_W = EPADDED // NS
NPAD = 10112
ROWS_PER_W = NPAD // NS
EP = EPADDED // 2


def _emb_body(x_ref, w_ref, b_ref, g_ref, bb_ref, o_ref, osp_ref):
    h = jnp.dot(x_ref[...], w_ref[...], preferred_element_type=jnp.float32)
    h = h + b_ref[...]
    mu = jnp.mean(h, axis=1, keepdims=True)
    var = jnp.mean((h - mu) ** 2, axis=1, keepdims=True)
    h = (h - mu) / jnp.sqrt(var + 1e-5) * g_ref[...] + bb_ref[...]
    h = jnp.maximum(h, 0.0)
    o_ref[...] = h
    osp_ref[0] = h[:, :HALF]
    osp_ref[1] = h[:, HALF:]


def _node_embed(x2, W_emb, b_emb, ln_g, ln_b):
    return pl.pallas_call(
        _emb_body,
        out_shape=[jax.ShapeDtypeStruct((N_NODES, HIDDEN), jnp.float32),
                   jax.ShapeDtypeStruct((2, N_NODES, HALF), jnp.float32)],
    )(x2, W_emb, b_emb, ln_g, ln_b)


_BE = 8192


def _edge_body(a_ref, w_ref, b_ref, o_ref):
    e = jnp.dot(a_ref[...], w_ref[0], preferred_element_type=jnp.float32)
    o_ref[0] = e + b_ref[0]


def _edge_embed(attr2, W2, b2):
    return pl.pallas_call(
        _edge_body,
        grid=(NC, EP // _BE),
        in_specs=[
            pl.BlockSpec((_BE, 32), lambda c, i: (i, 0)),
            pl.BlockSpec((1, 32, HIDDEN), lambda c, i: (c, 0, 0)),
            pl.BlockSpec((8, HIDDEN), lambda c, i: (c, 0)),
        ],
        out_specs=pl.BlockSpec((1, _BE, HIDDEN), lambda c, i: (c, i, 0)),
        out_shape=jax.ShapeDtypeStruct((NC, EP, HIDDEN), jnp.float32),
    )(attr2, W2, b2)


def _sc_aggregate_body(h_hbm, ep_hbm, src_hbm, dst_hbm, z_hbm, out_hbm,
                       acc, src_a, dst_a, src_b, dst_b,
                       hs_a, hs_b, e_a, e_b, out_v,
                       gsem_a, gsem_b, esem_a, esem_b, isem_a, isem_b):
    c = lax.axis_index("c")
    s = lax.axis_index("s")
    pltpu.sync_copy(z_hbm, acc.at[pl.ds(s * ROWS_PER_W, ROWS_PER_W)])
    plsc.subcore_barrier()

    e_base0 = s * E_PER_W
    ep0 = s * (E_PER_W // 2)
    coff_h = c * N_NODES

    def load_idx(i, sv, dv, isem):
        base = e_base0 + i * CHUNK
        pltpu.async_copy(src_hbm.at[pl.ds(base, CHUNK)], sv, isem)
        pltpu.async_copy(dst_hbm.at[pl.ds(base, CHUNK)], dv, isem)

    def wait_idx(i, sv, dv, isem):
        base = e_base0 + i * CHUNK
        pltpu.make_async_copy(src_hbm.at[pl.ds(base, CHUNK)], sv, isem).wait()
        pltpu.make_async_copy(dst_hbm.at[pl.ds(base, CHUNK)], dv, isem).wait()
        for k in range(CHUNK // L):
            sv[pl.ds(k * L, L)] = sv[pl.ds(k * L, L)] + coff_h

    def load(i, sv, hs, eb, gsem, esem):
        pltpu.async_copy(h_hbm.at[sv], hs, gsem)
        pltpu.async_copy(
            ep_hbm.at[c, pl.ds(ep0 + i * (CHUNK // 2), CHUNK // 2)], eb, esem)

    def wait_load(i, sv, hs, eb, gsem, esem):
        pltpu.make_async_copy(h_hbm.at[sv], hs, gsem).wait()
        pltpu.make_async_copy(
            ep_hbm.at[c, pl.ds(ep0 + i * (CHUNK // 2), CHUNK // 2)], eb, esem).wait()

    def compute_scatter(dv, hs, eb):
        nq = HALF // L

        def pair_body(jp, carry2):
            keys = [(2 * jp + half, half, q) for half in range(2)
                    for q in range(nq)]
            hv = [hs[j, pl.ds(q * L, L)] for (j, _, q) in keys]
            ev = [eb[jp, pl.ds(half * HALF + q * L, L)]
                  for (_, half, q) in keys]
            t = [a + b for a, b in zip(hv, ev)]
            t = [jnp.maximum(a, 0.0) for a in t]
            ms = [a + EPS for a in t]
            exs = [jnp.exp(a) for a in ms]
            prs = [a * b for a, b in zip(ms, exs)]
            for k, (j, _, q) in enumerate(keys):
                out_v[j, pl.ds(q * L, L)] = prs[k]
            for k, (j, _, q) in enumerate(keys):
                out_v[j, pl.ds(HALF + q * L, L)] = exs[k]
            return carry2

        lax.fori_loop(0, CHUNK // 2, pair_body, 0)
        pltpu.sync_copy(out_v, acc.at[dv], add=True)

    load_idx(0, src_a, dst_a, isem_a)
    wait_idx(0, src_a, dst_a, isem_a)
    load(0, src_a, hs_a, e_a, gsem_a, esem_a)
    load_idx(1, src_b, dst_b, isem_b)

    def body(t, carry):
        i0 = 2 * t
        i1 = i0 + 1
        wait_idx(i1, src_b, dst_b, isem_b)
        load(i1, src_b, hs_b, e_b, gsem_b, esem_b)
        wait_load(i0, src_a, hs_a, e_a, gsem_a, esem_a)
        compute_scatter(dst_a, hs_a, e_a)

        @pl.when(t < N_CHUNKS // 2 - 1)
        def _():
            load_idx(i0 + 2, src_a, dst_a, isem_a)

        wait_load(i1, src_b, hs_b, e_b, gsem_b, esem_b)
        compute_scatter(dst_b, hs_b, e_b)

        @pl.when(t < N_CHUNKS // 2 - 1)
        def _():
            load_idx(i1 + 2, src_b, dst_b, isem_b)
            wait_idx(i0 + 2, src_a, dst_a, isem_a)
            load(i0 + 2, src_a, hs_a, e_a, gsem_a, esem_a)

        return carry

    lax.fori_loop(0, N_CHUNKS // 2, body, 0)
    plsc.subcore_barrier()
    pltpu.sync_copy(
        acc.at[pl.ds(s * ROWS_PER_W, ROWS_PER_W)],
        out_hbm.at[pl.ds(c * NPAD + s * ROWS_PER_W, ROWS_PER_W)],
    )


@functools.lru_cache(maxsize=1)
def _make_sc_aggregate():
    return functools.partial(
        pl.kernel,
        out_type=jax.ShapeDtypeStruct((NC * NPAD, HIDDEN), jnp.float32),
        mesh=plsc.VectorSubcoreMesh(
            core_axis_name="c", subcore_axis_name="s",
            num_cores=NC, num_subcores=NS,
        ),
        compiler_params=pltpu.CompilerParams(use_tc_tiling_on_sc=False),
        scratch_types=[
            pltpu.VMEM_SHARED((NPAD, HIDDEN), jnp.float32),
            pltpu.VMEM((CHUNK,), jnp.int32),
            pltpu.VMEM((CHUNK,), jnp.int32),
            pltpu.VMEM((CHUNK,), jnp.int32),
            pltpu.VMEM((CHUNK,), jnp.int32),
            pltpu.VMEM((CHUNK, HALF), jnp.float32),
            pltpu.VMEM((CHUNK, HALF), jnp.float32),
            pltpu.VMEM((CHUNK // 2, HIDDEN), jnp.float32),
            pltpu.VMEM((CHUNK // 2, HIDDEN), jnp.float32),
            pltpu.VMEM((CHUNK, HIDDEN), jnp.float32),
            pltpu.SemaphoreType.DMA,
            pltpu.SemaphoreType.DMA,
            pltpu.SemaphoreType.DMA,
            pltpu.SemaphoreType.DMA,
            pltpu.SemaphoreType.DMA,
            pltpu.SemaphoreType.DMA,
        ],
    )(_sc_aggregate_body)


def _sc_aggregate(h, ep, src, dst, zrows):
    return _make_sc_aggregate()(h, ep, src, dst, zrows)


def _mlp_body(h_ref, acc_ref,
              w0, b0, w1, b1, w2, b2, w3, b3, w4, b4,
              g0, bb0, g1, bb1, g2, bb2, g3, bb3,
              wd, bd, o_ref):
    h = h_ref[...]
    a0 = acc_ref[0][:N_NODES]
    a1 = acc_ref[1][:N_NODES]
    p = jnp.concatenate([a0[:, :HALF], a1[:, :HALF]], axis=1)
    dn = jnp.concatenate([a0[:, HALF:], a1[:, HALF:]], axis=1)
    z = p / (dn + 1e-16) + h
    ws = (w0, w1, w2, w3, w4)
    bs = (b0, b1, b2, b3, b4)
    gs = (g0, g1, g2, g3)
    bbs = (bb0, bb1, bb2, bb3)
    for i in range(5):
        z = jnp.dot(z, ws[i][...], preferred_element_type=jnp.float32) + bs[i][...]
        if i < 4:
            mu = jnp.mean(z, axis=0, keepdims=True)
            var = jnp.mean((z - mu) ** 2, axis=0, keepdims=True)
            z = (z - mu) / jnp.sqrt(var + 1e-5) * gs[i][...] + bbs[i][...]
            z = jnp.maximum(z, 0.0)
    z = jnp.maximum(z, 0.0)
    z = jnp.dot(z, wd[...], preferred_element_type=jnp.float32) + bd[...]
    o_ref[...] = z


def _combine_mlp(h, acc, mlp_Ws, mlp_bs, bn_gs, bn_bs, W_dec, b_dec):
    args = [h, acc]
    for w, b in zip(mlp_Ws, mlp_bs):
        args += [w, b]
    for g, b in zip(bn_gs, bn_bs):
        args += [g, b]
    args += [W_dec, b_dec]
    return pl.pallas_call(
        _mlp_body,
        out_shape=jax.ShapeDtypeStruct((N_NODES, HIDDEN), jnp.float32),
    )(*args)


def kernel(x, edge_index, edge_attr, W_emb, b_emb, ln_g, ln_b, W_edge, b_edge,
           mlp_W0, mlp_b0, mlp_W1, mlp_b1, mlp_W2, mlp_b2, mlp_W3, mlp_b3,
           mlp_W4, mlp_b4, bn_g0, bn_b0, bn_g1, bn_b1, bn_g2, bn_b2, bn_g3,
           bn_b3, W_dec, b_dec):
    x2 = x.reshape(N_NODES, -1)
    h, hsp = _node_embed(x2, W_emb, b_emb, ln_g, ln_b)

    zero16 = jnp.zeros((16, HALF), jnp.float32)
    W2 = jnp.stack([
        jnp.concatenate([
            jnp.concatenate([W_edge[:, c * HALF:(c + 1) * HALF], zero16], axis=1),
            jnp.concatenate([zero16, W_edge[:, c * HALF:(c + 1) * HALF]], axis=1),
        ], axis=0)
        for c in range(NC)
    ])
    b2 = jnp.stack([
        jnp.concatenate([b_edge[c * HALF:(c + 1) * HALF]] * 2)
        for c in range(NC)
    ])
    b2 = jnp.broadcast_to(b2[:, None, :], (NC, 8, HIDDEN)).reshape(NC * 8, HIDDEN)
    attr2 = edge_attr.reshape(N_EDGES // 2, 32)
    ep = _edge_embed(attr2, W2, b2)
    npad_e = EPADDED - N_EDGES

    src = jnp.pad(edge_index[0], (0, npad_e))
    dst = jnp.pad(edge_index[1], (0, npad_e), constant_values=NPAD - 1)
    zrows = jnp.zeros((ROWS_PER_W, HIDDEN), jnp.float32)
    acc = _sc_aggregate(hsp.reshape(NC * N_NODES, HALF), ep, src, dst, zrows)
    z = _combine_mlp(h, acc.reshape(NC, NPAD, HIDDEN),
                     (mlp_W0, mlp_W1, mlp_W2, mlp_W3, mlp_W4),
                     (mlp_b0, mlp_b1, mlp_b2, mlp_b3, mlp_b4),
                     (bn_g0, bn_g1, bn_g2, bn_g3),
                     (bn_b0, bn_b1, bn_b2, bn_b3),
                     W_dec, b_dec)
    return z.reshape(N_NODES, HIDDEN // 2, 2)


# speedup vs baseline: 1.4561x; 1.0091x over previous
import functools

import jax
import jax.numpy as jnp
from jax import lax
from jax.experimental import pallas as pl
from jax.experimental.pallas import tpu as pltpu
from jax.experimental.pallas import tpu_sc as plsc

N_NODES = 10000
N_EDGES = 320000
HIDDEN = 128
HALF = HIDDEN // 2
EPS = 1e-7
NC = 2
NS = 16
L = 16
CHUNK = 128
EPADDED = 327680
N_CHUNKS = 160
E_PER_W = EPADDED // NS
NPAD = 10112
ROWS_PER_W = NPAD // NS
EP = EPADDED // 2


def _emb_body(x_ref, w_ref, b_ref, g_ref, bb_ref, o_ref, osp_ref):
    h = jnp.dot(x_ref[...], w_ref[...], preferred_element_type=jnp.float32)
    h = h + b_ref[...]
    mu = jnp.mean(h, axis=1, keepdims=True)
    var = jnp.mean((h - mu) ** 2, axis=1, keepdims=True)
    h = (h - mu) / jnp.sqrt(var + 1e-5) * g_ref[...] + bb_ref[...]
    h = jnp.maximum(h, 0.0)
    o_ref[...] = h
    osp_ref[0] = h[:, :HALF]
    osp_ref[1] = h[:, HALF:]


def _node_embed(x2, W_emb, b_emb, ln_g, ln_b):
    return pl.pallas_call(
        _emb_body,
        out_shape=[jax.ShapeDtypeStruct((N_NODES, HIDDEN), jnp.float32),
                   jax.ShapeDtypeStruct((2, N_NODES, HALF), jnp.float32)],
    )(x2, W_emb, b_emb, ln_g, ln_b)


_BE = 16384


def _edge_body(a_ref, w_ref, b_ref, o_ref):
    e = jnp.dot(a_ref[...], w_ref[0], preferred_element_type=jnp.float32)
    o_ref[0] = e + b_ref[0]


def _edge_embed(attr2, W2, b2):
    return pl.pallas_call(
        _edge_body,
        grid=(NC, EP // _BE),
        in_specs=[
            pl.BlockSpec((_BE, 32), lambda c, i: (i, 0)),
            pl.BlockSpec((1, 32, HIDDEN), lambda c, i: (c, 0, 0)),
            pl.BlockSpec((8, HIDDEN), lambda c, i: (c, 0)),
        ],
        out_specs=pl.BlockSpec((1, _BE, HIDDEN), lambda c, i: (c, i, 0)),
        out_shape=jax.ShapeDtypeStruct((NC, EP, HIDDEN), jnp.float32),
    )(attr2, W2, b2)


def _sc_aggregate_body(h_hbm, ep_hbm, src_hbm, dst_hbm, z_hbm, out_hbm,
                       acc, src_a, dst_a, src_b, dst_b,
                       hs_a, hs_b, e_a, e_b, out_v,
                       gsem_a, gsem_b, esem_a, esem_b, isem_a, isem_b):
    c = lax.axis_index("c")
    s = lax.axis_index("s")
    pltpu.sync_copy(z_hbm, acc.at[pl.ds(s * ROWS_PER_W, ROWS_PER_W)])
    plsc.subcore_barrier()

    e_base0 = s * E_PER_W
    ep0 = (c * EP + s * (E_PER_W // 2)) * HIDDEN
    coff_h = c * N_NODES

    def load_idx(i, sv, dv, isem):
        base = e_base0 + i * CHUNK
        pltpu.async_copy(src_hbm.at[pl.ds(base, CHUNK)], sv, isem)
        pltpu.async_copy(dst_hbm.at[pl.ds(base, CHUNK)], dv, isem)

    def wait_idx(i, sv, dv, isem):
        base = e_base0 + i * CHUNK
        pltpu.make_async_copy(src_hbm.at[pl.ds(base, CHUNK)], sv, isem).wait()
        pltpu.make_async_copy(dst_hbm.at[pl.ds(base, CHUNK)], dv, isem).wait()
        for k in range(CHUNK // L):
            sv[pl.ds(k * L, L)] = sv[pl.ds(k * L, L)] + coff_h

    def load(i, sv, hs, eb, gsem, esem):
        pltpu.async_copy(h_hbm.at[sv], hs, gsem)
        pltpu.async_copy(
            ep_hbm.at[pl.ds(ep0 + i * (CHUNK // 2) * HIDDEN,
                            (CHUNK // 2) * HIDDEN)], eb, esem)

    def wait_load(i, sv, hs, eb, gsem, esem):
        pltpu.make_async_copy(h_hbm.at[sv], hs, gsem).wait()
        pltpu.make_async_copy(
            ep_hbm.at[pl.ds(ep0 + i * (CHUNK // 2) * HIDDEN,
                            (CHUNK // 2) * HIDDEN)], eb, esem).wait()

    def compute_scatter(dv, hs, eb):
        nq = HALF // L

        def pair_body(jp, carry2):
            keys = [(2 * jp + half, half, q) for half in range(2)
                    for q in range(nq)]
            hv = [hs[j, pl.ds(q * L, L)] for (j, _, q) in keys]
            ev = [eb[pl.ds(jp * HIDDEN + half * HALF + q * L, L)]
                  for (_, half, q) in keys]
            t = [a + b for a, b in zip(hv, ev)]
            t = [jnp.maximum(a, 0.0) for a in t]
            ms = [a + EPS for a in t]
            exs = [jnp.exp(a) for a in ms]
            prs = [a * b for a, b in zip(ms, exs)]
            for k, (j, _, q) in enumerate(keys):
                out_v[j, pl.ds(q * L, L)] = prs[k]
            for k, (j, _, q) in enumerate(keys):
                out_v[j, pl.ds(HALF + q * L, L)] = exs[k]
            return carry2

        lax.fori_loop(0, CHUNK // 2, pair_body, 0)
        pltpu.sync_copy(out_v, acc.at[dv], add=True)

    load_idx(0, src_a, dst_a, isem_a)
    wait_idx(0, src_a, dst_a, isem_a)
    load(0, src_a, hs_a, e_a, gsem_a, esem_a)
    load_idx(1, src_b, dst_b, isem_b)

    def body(t, carry):
        i0 = 2 * t
        i1 = i0 + 1
        wait_idx(i1, src_b, dst_b, isem_b)
        load(i1, src_b, hs_b, e_b, gsem_b, esem_b)
        wait_load(i0, src_a, hs_a, e_a, gsem_a, esem_a)
        compute_scatter(dst_a, hs_a, e_a)

        @pl.when(t < N_CHUNKS // 2 - 1)
        def _():
            load_idx(i0 + 2, src_a, dst_a, isem_a)

        wait_load(i1, src_b, hs_b, e_b, gsem_b, esem_b)
        compute_scatter(dst_b, hs_b, e_b)

        @pl.when(t < N_CHUNKS // 2 - 1)
        def _():
            load_idx(i1 + 2, src_b, dst_b, isem_b)
            wait_idx(i0 + 2, src_a, dst_a, isem_a)
            load(i0 + 2, src_a, hs_a, e_a, gsem_a, esem_a)

        return carry

    lax.fori_loop(0, N_CHUNKS // 2, body, 0)
    plsc.subcore_barrier()
    pltpu.sync_copy(
        acc.at[pl.ds(s * ROWS_PER_W, ROWS_PER_W)],
        out_hbm.at[pl.ds(c * NPAD + s * ROWS_PER_W, ROWS_PER_W)],
    )


@functools.lru_cache(maxsize=1)
def _make_sc_aggregate():
    return functools.partial(
        pl.kernel,
        out_type=jax.ShapeDtypeStruct((NC * NPAD, HIDDEN), jnp.float32),
        mesh=plsc.VectorSubcoreMesh(
            core_axis_name="c", subcore_axis_name="s",
            num_cores=NC, num_subcores=NS,
        ),
        compiler_params=pltpu.CompilerParams(use_tc_tiling_on_sc=False),
        scratch_types=[
            pltpu.VMEM_SHARED((NPAD, HIDDEN), jnp.float32),
            pltpu.VMEM((CHUNK,), jnp.int32),
            pltpu.VMEM((CHUNK,), jnp.int32),
            pltpu.VMEM((CHUNK,), jnp.int32),
            pltpu.VMEM((CHUNK,), jnp.int32),
            pltpu.VMEM((CHUNK, HALF), jnp.float32),
            pltpu.VMEM((CHUNK, HALF), jnp.float32),
            pltpu.VMEM(((CHUNK // 2) * HIDDEN,), jnp.float32),
            pltpu.VMEM(((CHUNK // 2) * HIDDEN,), jnp.float32),
            pltpu.VMEM((CHUNK, HIDDEN), jnp.float32),
            pltpu.SemaphoreType.DMA,
            pltpu.SemaphoreType.DMA,
            pltpu.SemaphoreType.DMA,
            pltpu.SemaphoreType.DMA,
            pltpu.SemaphoreType.DMA,
            pltpu.SemaphoreType.DMA,
        ],
    )(_sc_aggregate_body)


def _sc_aggregate(h, ep, src, dst, zrows):
    return _make_sc_aggregate()(h, ep, src, dst, zrows)


def _mlp_body(h_ref, acc_ref,
              w0, b0, w1, b1, w2, b2, w3, b3, w4, b4,
              g0, bb0, g1, bb1, g2, bb2, g3, bb3,
              wd, bd, o_ref):
    h = h_ref[...]
    a0 = acc_ref[0][:N_NODES]
    a1 = acc_ref[1][:N_NODES]
    p = jnp.concatenate([a0[:, :HALF], a1[:, :HALF]], axis=1)
    dn = jnp.concatenate([a0[:, HALF:], a1[:, HALF:]], axis=1)
    z = p / (dn + 1e-16) + h
    ws = (w0, w1, w2, w3, w4)
    bs = (b0, b1, b2, b3, b4)
    gs = (g0, g1, g2, g3)
    bbs = (bb0, bb1, bb2, bb3)
    for i in range(5):
        z = jnp.dot(z, ws[i][...], preferred_element_type=jnp.float32) + bs[i][...]
        if i < 4:
            mu = jnp.mean(z, axis=0, keepdims=True)
            var = jnp.mean((z - mu) ** 2, axis=0, keepdims=True)
            z = (z - mu) / jnp.sqrt(var + 1e-5) * gs[i][...] + bbs[i][...]
            z = jnp.maximum(z, 0.0)
    z = jnp.maximum(z, 0.0)
    z = jnp.dot(z, wd[...], preferred_element_type=jnp.float32) + bd[...]
    o_ref[...] = z


def _combine_mlp(h, acc, mlp_Ws, mlp_bs, bn_gs, bn_bs, W_dec, b_dec):
    args = [h, acc]
    for w, b in zip(mlp_Ws, mlp_bs):
        args += [w, b]
    for g, b in zip(bn_gs, bn_bs):
        args += [g, b]
    args += [W_dec, b_dec]
    return pl.pallas_call(
        _mlp_body,
        out_shape=jax.ShapeDtypeStruct((N_NODES, HIDDEN), jnp.float32),
    )(*args)


def kernel(x, edge_index, edge_attr, W_emb, b_emb, ln_g, ln_b, W_edge, b_edge,
           mlp_W0, mlp_b0, mlp_W1, mlp_b1, mlp_W2, mlp_b2, mlp_W3, mlp_b3,
           mlp_W4, mlp_b4, bn_g0, bn_b0, bn_g1, bn_b1, bn_g2, bn_b2, bn_g3,
           bn_b3, W_dec, b_dec):
    x2 = x.reshape(N_NODES, -1)
    h, hsp = _node_embed(x2, W_emb, b_emb, ln_g, ln_b)

    zero16 = jnp.zeros((16, HALF), jnp.float32)
    W2 = jnp.stack([
        jnp.concatenate([
            jnp.concatenate([W_edge[:, c * HALF:(c + 1) * HALF], zero16], axis=1),
            jnp.concatenate([zero16, W_edge[:, c * HALF:(c + 1) * HALF]], axis=1),
        ], axis=0)
        for c in range(NC)
    ])
    b2 = jnp.stack([
        jnp.concatenate([b_edge[c * HALF:(c + 1) * HALF]] * 2)
        for c in range(NC)
    ])
    b2 = jnp.broadcast_to(b2[:, None, :], (NC, 8, HIDDEN)).reshape(NC * 8, HIDDEN)
    attr2 = edge_attr.reshape(N_EDGES // 2, 32)
    ep = _edge_embed(attr2, W2, b2)
    npad_e = EPADDED - N_EDGES

    src = jnp.pad(edge_index[0], (0, npad_e))
    dst = jnp.pad(edge_index[1], (0, npad_e), constant_values=NPAD - 1)
    zrows = jnp.zeros((ROWS_PER_W, HIDDEN), jnp.float32)
    acc = _sc_aggregate(hsp.reshape(NC * N_NODES, HALF),
                        ep.reshape(NC * EP * HIDDEN), src, dst, zrows)
    z = _combine_mlp(h, acc.reshape(NC, NPAD, HIDDEN),
                     (mlp_W0, mlp_W1, mlp_W2, mlp_W3, mlp_W4),
                     (mlp_b0, mlp_b1, mlp_b2, mlp_b3, mlp_b4),
                     (bn_g0, bn_g1, bn_g2, bn_g3),
                     (bn_b0, bn_b1, bn_b2, bn_b3),
                     W_dec, b_dec)
    return z.reshape(N_NODES, HIDDEN // 2, 2)
